# baseline (speedup 1.0000x reference)
import jax
import jax.numpy as jnp
from jax import lax
from jax.experimental import pallas as pl
from jax.experimental.pallas import tpu as pltpu


def _rbm_kernel(x_ref, q_ref, ei_ref, ej_ref, lin_ref, out_ref, qmat_ref):
    n = x_ref.shape[1]
    e = q_ref.shape[1]

    @pl.when(pl.program_id(0) == 0)
    def _build_q():
        node_iota = lax.broadcasted_iota(jnp.int32, (n, e), 0)
        mi = (node_iota == ei_ref[:, :]).astype(jnp.float32)
        mj = (node_iota == ej_ref[:, :]).astype(jnp.float32)
        qmat_ref[:, :] = lax.dot_general(
            mi * q_ref[:, :], mj,
            dimension_numbers=(((1,), (1,)), ((), ())),
            preferred_element_type=jnp.float32,
        )

    xb = x_ref[:, :]
    y = jnp.dot(xb, qmat_ref[:, :], preferred_element_type=jnp.float32)
    y = y + lin_ref[:, :]
    out_ref[:, :] = jnp.sum(xb * y, axis=1, keepdims=True)


def kernel(x, linear, quadratic, edge_idx_i, edge_idx_j):
    batch, n = x.shape
    e = quadratic.shape[0]
    q2 = quadratic.astype(jnp.float32).reshape(1, e)
    ei = edge_idx_i.astype(jnp.int32).reshape(1, e)
    ej = edge_idx_j.astype(jnp.int32).reshape(1, e)
    lin = linear.astype(jnp.float32).reshape(1, n)

    blk = 8192
    out = pl.pallas_call(
        _rbm_kernel,
        grid=(batch // blk,),
        in_specs=[
            pl.BlockSpec((blk, n), lambda i: (i, 0)),
            pl.BlockSpec((1, e), lambda i: (0, 0)),
            pl.BlockSpec((1, e), lambda i: (0, 0)),
            pl.BlockSpec((1, e), lambda i: (0, 0)),
            pl.BlockSpec((1, n), lambda i: (0, 0)),
        ],
        out_specs=pl.BlockSpec((blk, 1), lambda i: (i, 0)),
        out_shape=jax.ShapeDtypeStruct((batch, 1), jnp.float32),
        scratch_shapes=[pltpu.VMEM((n, n), jnp.float32)],
        compiler_params=pltpu.CompilerParams(
            dimension_semantics=("arbitrary",),
        ),
    )(x, q2, ei, ej, lin)
    return out.reshape(batch)
